# baseline (speedup 1.0000x reference)
import functools

import jax
import jax.numpy as jnp
from jax import lax
from jax.experimental import pallas as pl
from jax.experimental.pallas import tpu as pltpu

LNC = 5000
DIS = 3000
N = LNC + DIS
IN_C = 128
OUT_C = 64
N_HEAD = 4
NEG_SLOPE = 0.2

NP = 8192
BP = 512
BS = 512
BD = 512
FA = N_HEAD * OUT_C
AUGC = FA + 8


def _prologue_body(z_ref, w_ref, al_ref, ar_ref,
                   feat_ref, a_ref, a2_ref, ct_ref):
    z = z_ref[...]
    featf = jnp.dot(z, w_ref[...], preferred_element_type=jnp.float32)
    el = jnp.dot(featf, al_ref[...], preferred_element_type=jnp.float32)
    ert = lax.dot_general(
        ar_ref[...], featf, (((0,), (1,)), ((), ())),
        preferred_element_type=jnp.float32)
    a_ref[...] = jnp.exp(el)
    a2_ref[...] = jnp.exp(NEG_SLOPE * el)
    ct_ref[...] = jnp.exp((1.0 - NEG_SLOPE) * ert)
    fb = featf.astype(jnp.bfloat16)
    ones = jnp.ones((BP, 8), jnp.bfloat16)
    feat_ref[...] = jnp.concatenate([fb, ones], axis=1)


def _main_body(adj_ref, feat_ref, a_ref, a2_ref, ct_ref, bias_ref, out_ref,
               acc_ref, den_ref, *, ns):
    s = pl.program_id(1)

    @pl.when(s == 0)
    def _init():
        acc_ref[...] = jnp.zeros_like(acc_ref)
        den_ref[...] = jnp.zeros_like(den_ref)

    adj = adj_ref[...]
    row = lax.broadcasted_iota(jnp.int32, (BS, 1), 0) + s * BS
    edge = (adj == 1) & (row < N)

    feat = feat_ref[pl.ds(s * BS, BS), :]
    a = a_ref[...]
    a2 = a2_ref[...]
    ct = ct_ref[...]
    ones_col = feat[:, FA:FA + 8]
    for h in range(N_HEAD):
        q = jnp.maximum(a[:, h:h + 1] * ct[h:h + 1, :], a2[:, h:h + 1])
        p = jnp.where(edge, q, 0.0).astype(jnp.bfloat16)
        acc_ref[:, h * OUT_C:(h + 1) * OUT_C] += lax.dot_general(
            p, feat[:, h * OUT_C:(h + 1) * OUT_C],
            (((0,), (0,)), ((), ())), preferred_element_type=jnp.float32)
        den_ref[:, h * 8:(h + 1) * 8] += lax.dot_general(
            p, ones_col, (((0,), (0,)), ((), ())),
            preferred_element_type=jnp.float32)

    @pl.when(s == ns - 1)
    def _finalize():
        parts = []
        for h in range(N_HEAD):
            d = den_ref[:, h * 8:h * 8 + 1]
            parts.append(acc_ref[:, h * OUT_C:(h + 1) * OUT_C]
                         / jnp.maximum(d, 1e-16))
        r = jnp.concatenate(parts, axis=1) + bias_ref[...]
        out_ref[...] = jnp.where(r > 0, r, jnp.exp(r) - 1.0)


@jax.jit
def kernel(lncrna_x, disease_x, adj, W, attn_l, attn_r, bias):
    z = jnp.concatenate([lncrna_x, disease_x], axis=0)
    zp = jnp.pad(z, ((0, NP - N), (0, 0)))

    eye = jnp.eye(N_HEAD, 8, dtype=jnp.float32)
    a_l = (attn_l[:, :, None] * eye[:, None, :]).reshape(FA, 8)
    a_r = (attn_r[:, :, None] * eye[:, None, :]).reshape(FA, 8)

    feat, a_e, a2_e, ct_e = pl.pallas_call(
        _prologue_body,
        grid=(NP // BP,),
        in_specs=[
            pl.BlockSpec((BP, IN_C), lambda i: (i, 0)),
            pl.BlockSpec((IN_C, FA), lambda i: (0, 0)),
            pl.BlockSpec((FA, 8), lambda i: (0, 0)),
            pl.BlockSpec((FA, 8), lambda i: (0, 0)),
        ],
        out_specs=[
            pl.BlockSpec((BP, AUGC), lambda i: (i, 0)),
            pl.BlockSpec((BP, 8), lambda i: (i, 0)),
            pl.BlockSpec((BP, 8), lambda i: (i, 0)),
            pl.BlockSpec((8, BP), lambda i: (0, i)),
        ],
        out_shape=[
            jax.ShapeDtypeStruct((NP, AUGC), jnp.bfloat16),
            jax.ShapeDtypeStruct((NP, 8), jnp.float32),
            jax.ShapeDtypeStruct((NP, 8), jnp.float32),
            jax.ShapeDtypeStruct((8, NP), jnp.float32),
        ],
    )(zp, W, a_l, a_r)

    nd, ns = NP // BD, NP // BS
    out = pl.pallas_call(
        functools.partial(_main_body, ns=ns),
        grid=(nd, ns),
        in_specs=[
            pl.BlockSpec((BS, BD), lambda d, s: (s, d)),
            pl.BlockSpec((NP, AUGC), lambda d, s: (0, 0)),
            pl.BlockSpec((BS, 8), lambda d, s: (s, 0)),
            pl.BlockSpec((BS, 8), lambda d, s: (s, 0)),
            pl.BlockSpec((8, BD), lambda d, s: (0, d)),
            pl.BlockSpec((1, FA), lambda d, s: (0, 0)),
        ],
        out_specs=pl.BlockSpec((BD, FA), lambda d, s: (d, 0)),
        out_shape=jax.ShapeDtypeStruct((NP, FA), jnp.float32),
        scratch_shapes=[
            pltpu.VMEM((BD, FA), jnp.float32),
            pltpu.VMEM((BD, 32), jnp.float32),
        ],
        compiler_params=pltpu.CompilerParams(
            dimension_semantics=("parallel", "arbitrary")),
    )(adj, feat, a_e, a2_e, ct_e, bias.reshape(1, FA))
    return out[:N]


# speedup vs baseline: 1.3844x; 1.3844x over previous
import functools

import jax
import jax.numpy as jnp
from jax import lax
from jax.experimental import pallas as pl
from jax.experimental.pallas import tpu as pltpu

LNC = 5000
DIS = 3000
N = LNC + DIS
IN_C = 128
OUT_C = 64
N_HEAD = 4
NEG_SLOPE = 0.2

NP = 8192
BP = 512
BS = 512
BD = 512
FA = N_HEAD * OUT_C
HG = OUT_C + 8
AUGC = N_HEAD * HG


def _prologue_body(z_ref, w_ref, al_ref, ar_ref,
                   feat_ref, a8_ref, ct_ref):
    z = z_ref[...]
    featf = jnp.dot(z, w_ref[...], preferred_element_type=jnp.float32)
    el = jnp.dot(featf, al_ref[...], preferred_element_type=jnp.float32)
    ert = lax.dot_general(
        ar_ref[...], featf, (((0,), (1,)), ((), ())),
        preferred_element_type=jnp.float32)
    a8_ref[...] = jnp.exp((1.0 - NEG_SLOPE) * el).astype(jnp.bfloat16)
    ct_ref[...] = jnp.exp((1.0 - NEG_SLOPE) * ert).astype(jnp.bfloat16)
    a2 = jnp.exp(NEG_SLOPE * el)
    parts = []
    for h in range(N_HEAD):
        a2h = a2[:, h:h + 1]
        parts.append(featf[:, h * OUT_C:(h + 1) * OUT_C] * a2h)
        parts.append(jnp.broadcast_to(a2h, (BP, 8)))
    feat_ref[...] = jnp.concatenate(parts, axis=1).astype(jnp.bfloat16)


def _main_body(adj_ref, feat_ref, a8_ref, ct_ref, bias_ref, out_ref,
               acc_ref, *, ns):
    s = pl.program_id(1)

    @pl.when(s == 0)
    def _init():
        acc_ref[...] = jnp.zeros_like(acc_ref)

    adj = adj_ref[...]
    row = lax.broadcasted_iota(jnp.int32, (BS, 1), 0) + s * BS
    edge = (adj == 1) & (row < N)
    maskb = jnp.where(edge, 1.0, 0.0).astype(jnp.bfloat16)

    feat = feat_ref[pl.ds(s * BS, BS), :]
    a8 = a8_ref[...]
    ct = ct_ref[...]
    one = jnp.bfloat16(1.0)
    for h in range(N_HEAD):
        t = a8[:, h:h + 1] * ct[h:h + 1, :]
        p = jnp.maximum(t, one) * maskb
        acc_ref[:, h * HG:(h + 1) * HG] += lax.dot_general(
            p, feat[:, h * HG:(h + 1) * HG],
            (((0,), (0,)), ((), ())), preferred_element_type=jnp.float32)

    @pl.when(s == ns - 1)
    def _finalize():
        parts = []
        for h in range(N_HEAD):
            d = acc_ref[:, h * HG + OUT_C:h * HG + OUT_C + 1]
            parts.append(acc_ref[:, h * HG:h * HG + OUT_C]
                         / jnp.maximum(d, 1e-16))
        r = jnp.concatenate(parts, axis=1) + bias_ref[...]
        out_ref[...] = jnp.where(r > 0, r, jnp.exp(r) - 1.0)


@jax.jit
def kernel(lncrna_x, disease_x, adj, W, attn_l, attn_r, bias):
    z = jnp.concatenate([lncrna_x, disease_x], axis=0)
    zp = jnp.pad(z, ((0, NP - N), (0, 0)))

    eye = jnp.eye(N_HEAD, 8, dtype=jnp.float32)
    a_l = (attn_l[:, :, None] * eye[:, None, :]).reshape(FA, 8)
    a_r = (attn_r[:, :, None] * eye[:, None, :]).reshape(FA, 8)

    feat, a8_e, ct_e = pl.pallas_call(
        _prologue_body,
        grid=(NP // BP,),
        in_specs=[
            pl.BlockSpec((BP, IN_C), lambda i: (i, 0)),
            pl.BlockSpec((IN_C, FA), lambda i: (0, 0)),
            pl.BlockSpec((FA, 8), lambda i: (0, 0)),
            pl.BlockSpec((FA, 8), lambda i: (0, 0)),
        ],
        out_specs=[
            pl.BlockSpec((BP, AUGC), lambda i: (i, 0)),
            pl.BlockSpec((BP, 8), lambda i: (i, 0)),
            pl.BlockSpec((8, BP), lambda i: (0, i)),
        ],
        out_shape=[
            jax.ShapeDtypeStruct((NP, AUGC), jnp.bfloat16),
            jax.ShapeDtypeStruct((NP, 8), jnp.bfloat16),
            jax.ShapeDtypeStruct((8, NP), jnp.bfloat16),
        ],
    )(zp, W, a_l, a_r)

    nd, ns = NP // BD, NP // BS
    out = pl.pallas_call(
        functools.partial(_main_body, ns=ns),
        grid=(nd, ns),
        in_specs=[
            pl.BlockSpec((BS, BD), lambda d, s: (s, d)),
            pl.BlockSpec((NP, AUGC), lambda d, s: (0, 0)),
            pl.BlockSpec((BS, 8), lambda d, s: (s, 0)),
            pl.BlockSpec((8, BD), lambda d, s: (0, d)),
            pl.BlockSpec((1, FA), lambda d, s: (0, 0)),
        ],
        out_specs=pl.BlockSpec((BD, FA), lambda d, s: (d, 0)),
        out_shape=jax.ShapeDtypeStruct((NP, FA), jnp.float32),
        scratch_shapes=[
            pltpu.VMEM((BD, AUGC), jnp.float32),
        ],
        compiler_params=pltpu.CompilerParams(
            dimension_semantics=("parallel", "arbitrary")),
    )(adj, feat, a8_e, ct_e, bias.reshape(1, FA))
    return out[:N]


# speedup vs baseline: 1.4511x; 1.0482x over previous
import functools

import jax
import jax.numpy as jnp
from jax import lax
from jax.experimental import pallas as pl
from jax.experimental.pallas import tpu as pltpu

LNC = 5000
DIS = 3000
N = LNC + DIS
IN_C = 128
OUT_C = 64
N_HEAD = 4
NEG_SLOPE = 0.2

NP = 8192
BP = 512
BS = 512
BD = 512
FA = N_HEAD * OUT_C
HG = OUT_C + 8
AUGC = N_HEAD * HG


def _prologue_body(z_ref, w_ref, al_ref, ar_ref,
                   feat_ref, a8_ref, ct_ref):
    z = z_ref[...]
    featf = jnp.dot(z, w_ref[...], preferred_element_type=jnp.float32)
    el = jnp.dot(featf, al_ref[...], preferred_element_type=jnp.float32)
    ert = lax.dot_general(
        ar_ref[...], featf, (((0,), (1,)), ((), ())),
        preferred_element_type=jnp.float32)
    a8 = jnp.exp((1.0 - NEG_SLOPE) * el)
    ct = jnp.exp((1.0 - NEG_SLOPE) * ert)
    zc = jnp.zeros((BP, 7), jnp.float32)
    a8_parts = []
    for h in range(N_HEAD):
        a8_parts.append(a8[:, h:h + 1])
        a8_parts.append(zc)
    a8_ref[...] = jnp.concatenate(a8_parts, axis=1).astype(jnp.bfloat16)
    zr = jnp.zeros((7, BP), jnp.float32)
    ct_parts = []
    for h in range(N_HEAD):
        ct_parts.append(ct[h:h + 1, :])
        ct_parts.append(zr)
    ct_ref[...] = jnp.concatenate(ct_parts, axis=0).astype(jnp.bfloat16)
    a2 = jnp.exp(NEG_SLOPE * el)
    parts = []
    for h in range(N_HEAD):
        a2h = a2[:, h:h + 1]
        parts.append(featf[:, h * OUT_C:(h + 1) * OUT_C] * a2h)
        parts.append(jnp.broadcast_to(a2h, (BP, 8)))
    feat_ref[...] = jnp.concatenate(parts, axis=1).astype(jnp.bfloat16)


def _main_body(adj_ref, feat_ref, a8_ref, ct_ref, bias_ref, out_ref,
               acc_ref, *, ns):
    s = pl.program_id(1)

    @pl.when(s == 0)
    def _init():
        acc_ref[...] = jnp.zeros_like(acc_ref)

    adj = adj_ref[...]
    row = lax.broadcasted_iota(jnp.int32, (BS, 1), 0) + s * BS
    edge = (adj == 1) & (row < N)
    maskb = jnp.where(edge, 1.0, 0.0).astype(jnp.bfloat16)

    feat = feat_ref[pl.ds(s * BS, BS), :]
    a8 = a8_ref[...]
    ct = ct_ref[...]
    one = jnp.bfloat16(1.0)
    for h in range(N_HEAD):
        t = lax.dot_general(
            a8[:, 8 * h:8 * h + 8], ct[8 * h:8 * h + 8, :],
            (((1,), (0,)), ((), ())),
            preferred_element_type=jnp.float32).astype(jnp.bfloat16)
        p = jnp.maximum(t, one) * maskb
        acc_ref[:, h * HG:(h + 1) * HG] += lax.dot_general(
            p, feat[:, h * HG:(h + 1) * HG],
            (((0,), (0,)), ((), ())), preferred_element_type=jnp.float32)

    @pl.when(s == ns - 1)
    def _finalize():
        parts = []
        for h in range(N_HEAD):
            d = acc_ref[:, h * HG + OUT_C:h * HG + OUT_C + 1]
            parts.append(acc_ref[:, h * HG:h * HG + OUT_C]
                         / jnp.maximum(d, 1e-16))
        r = jnp.concatenate(parts, axis=1) + bias_ref[...]
        out_ref[...] = jnp.where(r > 0, r, jnp.exp(r) - 1.0)


@jax.jit
def kernel(lncrna_x, disease_x, adj, W, attn_l, attn_r, bias):
    z = jnp.concatenate([lncrna_x, disease_x], axis=0)
    zp = jnp.pad(z, ((0, NP - N), (0, 0)))

    eye = jnp.eye(N_HEAD, 8, dtype=jnp.float32)
    a_l = (attn_l[:, :, None] * eye[:, None, :]).reshape(FA, 8)
    a_r = (attn_r[:, :, None] * eye[:, None, :]).reshape(FA, 8)

    feat, a8_e, ct_e = pl.pallas_call(
        _prologue_body,
        grid=(NP // BP,),
        in_specs=[
            pl.BlockSpec((BP, IN_C), lambda i: (i, 0)),
            pl.BlockSpec((IN_C, FA), lambda i: (0, 0)),
            pl.BlockSpec((FA, 8), lambda i: (0, 0)),
            pl.BlockSpec((FA, 8), lambda i: (0, 0)),
        ],
        out_specs=[
            pl.BlockSpec((BP, AUGC), lambda i: (i, 0)),
            pl.BlockSpec((BP, 32), lambda i: (i, 0)),
            pl.BlockSpec((32, BP), lambda i: (0, i)),
        ],
        out_shape=[
            jax.ShapeDtypeStruct((NP, AUGC), jnp.bfloat16),
            jax.ShapeDtypeStruct((NP, 32), jnp.bfloat16),
            jax.ShapeDtypeStruct((32, NP), jnp.bfloat16),
        ],
    )(zp, W, a_l, a_r)

    nd, ns = NP // BD, NP // BS
    out = pl.pallas_call(
        functools.partial(_main_body, ns=ns),
        grid=(nd, ns),
        in_specs=[
            pl.BlockSpec((BS, BD), lambda d, s: (s, d)),
            pl.BlockSpec((NP, AUGC), lambda d, s: (0, 0)),
            pl.BlockSpec((BS, 32), lambda d, s: (s, 0)),
            pl.BlockSpec((32, BD), lambda d, s: (0, d)),
            pl.BlockSpec((1, FA), lambda d, s: (0, 0)),
        ],
        out_specs=pl.BlockSpec((BD, FA), lambda d, s: (d, 0)),
        out_shape=jax.ShapeDtypeStruct((NP, FA), jnp.float32),
        scratch_shapes=[
            pltpu.VMEM((BD, AUGC), jnp.float32),
        ],
        compiler_params=pltpu.CompilerParams(
            dimension_semantics=("parallel", "arbitrary")),
    )(adj, feat, a8_e, ct_e, bias.reshape(1, FA))
    return out[:N]


# speedup vs baseline: 1.8032x; 1.2426x over previous
import functools

import jax
import jax.numpy as jnp
from jax import lax
from jax.experimental import pallas as pl
from jax.experimental.pallas import tpu as pltpu

LNC = 5000
DIS = 3000
N = LNC + DIS
IN_C = 128
OUT_C = 64
N_HEAD = 4
NEG_SLOPE = 0.2

NP = 8192
BP = 512
BS = 1024
BD = 512
FA = N_HEAD * OUT_C
HG = OUT_C + 8
AUGC = N_HEAD * HG


def _prologue_body(z_ref, w_ref, al_ref, ar_ref,
                   feat_ref, a8_ref, ct_ref):
    i = pl.program_id(0)
    z = z_ref[...]
    featf = jnp.dot(z, w_ref[...], preferred_element_type=jnp.float32)
    el = jnp.dot(featf, al_ref[...], preferred_element_type=jnp.float32)
    ert = lax.dot_general(
        ar_ref[...], featf, (((0,), (1,)), ((), ())),
        preferred_element_type=jnp.float32)
    a8 = jnp.exp((1.0 - NEG_SLOPE) * el)
    ct = jnp.exp((1.0 - NEG_SLOPE) * ert)
    zc = jnp.zeros((BP, 7), jnp.float32)
    a8_parts = []
    for h in range(N_HEAD):
        a8_parts.append(a8[:, h:h + 1])
        a8_parts.append(zc)
    a8_ref[...] = jnp.concatenate(a8_parts, axis=1).astype(jnp.bfloat16)
    zr = jnp.zeros((7, BP), jnp.float32)
    ct_parts = []
    for h in range(N_HEAD):
        ct_parts.append(ct[h:h + 1, :])
        ct_parts.append(zr)
    ct_ref[...] = jnp.concatenate(ct_parts, axis=0).astype(jnp.bfloat16)
    rowid = lax.broadcasted_iota(jnp.int32, (BP, 1), 0) + i * BP
    valid = (rowid < N).astype(jnp.float32)
    a2 = jnp.exp(NEG_SLOPE * el) * valid
    parts = []
    for h in range(N_HEAD):
        a2h = a2[:, h:h + 1]
        parts.append(featf[:, h * OUT_C:(h + 1) * OUT_C] * a2h)
        parts.append(jnp.broadcast_to(a2h, (BP, 8)))
    feat_ref[...] = jnp.concatenate(parts, axis=1).astype(jnp.bfloat16)


def _main_body(adj_ref, feat_ref, a8_ref, ct_ref, bias_ref, out_ref,
               acc_ref, *, ns):
    s = pl.program_id(1)

    @pl.when(s == 0)
    def _init():
        acc_ref[...] = jnp.zeros_like(acc_ref)

    adj = adj_ref[...]
    maskb = jnp.where(adj == 1, 1.0, 0.0).astype(jnp.bfloat16)

    feat = feat_ref[pl.ds(s * BS, BS), :]
    a8 = a8_ref[...]
    ct = ct_ref[...]
    one = jnp.bfloat16(1.0)
    for h in range(N_HEAD):
        t = lax.dot_general(
            a8[:, 8 * h:8 * h + 8], ct[8 * h:8 * h + 8, :],
            (((1,), (0,)), ((), ())),
            preferred_element_type=jnp.float32).astype(jnp.bfloat16)
        p = jnp.maximum(t, one) * maskb
        acc_ref[:, h * HG:(h + 1) * HG] += lax.dot_general(
            p, feat[:, h * HG:(h + 1) * HG],
            (((0,), (0,)), ((), ())), preferred_element_type=jnp.float32)

    @pl.when(s == ns - 1)
    def _finalize():
        parts = []
        for h in range(N_HEAD):
            d = acc_ref[:, h * HG + OUT_C:h * HG + OUT_C + 1]
            parts.append(acc_ref[:, h * HG:h * HG + OUT_C]
                         / jnp.maximum(d, 1e-16))
        r = jnp.concatenate(parts, axis=1) + bias_ref[...]
        out_ref[...] = jnp.where(r > 0, r, jnp.exp(r) - 1.0)


@jax.jit
def kernel(lncrna_x, disease_x, adj, W, attn_l, attn_r, bias):
    z = jnp.concatenate([lncrna_x, disease_x], axis=0)
    zp = jnp.pad(z, ((0, NP - N), (0, 0)))

    eye = jnp.eye(N_HEAD, 8, dtype=jnp.float32)
    a_l = (attn_l[:, :, None] * eye[:, None, :]).reshape(FA, 8)
    a_r = (attn_r[:, :, None] * eye[:, None, :]).reshape(FA, 8)

    feat, a8_e, ct_e = pl.pallas_call(
        _prologue_body,
        grid=(NP // BP,),
        in_specs=[
            pl.BlockSpec((BP, IN_C), lambda i: (i, 0)),
            pl.BlockSpec((IN_C, FA), lambda i: (0, 0)),
            pl.BlockSpec((FA, 8), lambda i: (0, 0)),
            pl.BlockSpec((FA, 8), lambda i: (0, 0)),
        ],
        out_specs=[
            pl.BlockSpec((BP, AUGC), lambda i: (i, 0)),
            pl.BlockSpec((BP, 32), lambda i: (i, 0)),
            pl.BlockSpec((32, BP), lambda i: (0, i)),
        ],
        out_shape=[
            jax.ShapeDtypeStruct((NP, AUGC), jnp.bfloat16),
            jax.ShapeDtypeStruct((NP, 32), jnp.bfloat16),
            jax.ShapeDtypeStruct((32, NP), jnp.bfloat16),
        ],
    )(zp, W, a_l, a_r)

    nd, ns = NP // BD, NP // BS
    out = pl.pallas_call(
        functools.partial(_main_body, ns=ns),
        grid=(nd, ns),
        in_specs=[
            pl.BlockSpec((BS, BD), lambda d, s: (s, d)),
            pl.BlockSpec((NP, AUGC), lambda d, s: (0, 0)),
            pl.BlockSpec((BS, 32), lambda d, s: (s, 0)),
            pl.BlockSpec((32, BD), lambda d, s: (0, d)),
            pl.BlockSpec((1, FA), lambda d, s: (0, 0)),
        ],
        out_specs=pl.BlockSpec((BD, FA), lambda d, s: (d, 0)),
        out_shape=jax.ShapeDtypeStruct((NP, FA), jnp.float32),
        scratch_shapes=[
            pltpu.VMEM((BD, AUGC), jnp.float32),
        ],
        compiler_params=pltpu.CompilerParams(
            dimension_semantics=("parallel", "arbitrary")),
    )(adj, feat, a8_e, ct_e, bias.reshape(1, FA))
    return out[:N]


# speedup vs baseline: 1.9359x; 1.0736x over previous
import functools

import jax
import jax.numpy as jnp
from jax import lax
from jax.experimental import pallas as pl
from jax.experimental.pallas import tpu as pltpu

LNC = 5000
DIS = 3000
N = LNC + DIS
IN_C = 128
OUT_C = 64
N_HEAD = 4
NEG_SLOPE = 0.2

NP = 8192
BP = 512
BS = 1024
BD = 1024
FA = N_HEAD * OUT_C
HG = OUT_C + 8
AUGC = N_HEAD * HG


def _prologue_body(z_ref, w_ref, al_ref, ar_ref,
                   feat_ref, a8_ref, ct_ref):
    i = pl.program_id(0)
    z = z_ref[...]
    featf = jnp.dot(z, w_ref[...], preferred_element_type=jnp.float32)
    el = jnp.dot(featf, al_ref[...], preferred_element_type=jnp.float32)
    ert = lax.dot_general(
        ar_ref[...], featf, (((0,), (1,)), ((), ())),
        preferred_element_type=jnp.float32)
    a8 = jnp.exp((1.0 - NEG_SLOPE) * el)
    ct = jnp.exp((1.0 - NEG_SLOPE) * ert)
    zc = jnp.zeros((BP, 7), jnp.float32)
    a8_parts = []
    for h in range(N_HEAD):
        a8_parts.append(a8[:, h:h + 1])
        a8_parts.append(zc)
    a8_ref[...] = jnp.concatenate(a8_parts, axis=1).astype(jnp.bfloat16)
    zr = jnp.zeros((7, BP), jnp.float32)
    ct_parts = []
    for h in range(N_HEAD):
        ct_parts.append(ct[h:h + 1, :])
        ct_parts.append(zr)
    ct_ref[...] = jnp.concatenate(ct_parts, axis=0).astype(jnp.bfloat16)
    rowid = lax.broadcasted_iota(jnp.int32, (BP, 1), 0) + i * BP
    valid = (rowid < N).astype(jnp.float32)
    a2 = jnp.exp(NEG_SLOPE * el) * valid
    parts = []
    for h in range(N_HEAD):
        a2h = a2[:, h:h + 1]
        parts.append(featf[:, h * OUT_C:(h + 1) * OUT_C] * a2h)
        parts.append(jnp.broadcast_to(a2h, (BP, 8)))
    feat_ref[...] = jnp.concatenate(parts, axis=1).astype(jnp.bfloat16)


def _main_body(adj_ref, feat_ref, a8_ref, ct_ref, bias_ref, out_ref,
               acc_ref, *, ns):
    s = pl.program_id(1)

    @pl.when(s == 0)
    def _init():
        acc_ref[...] = jnp.zeros_like(acc_ref)

    adj = adj_ref[...]
    maskb = jnp.where(adj == 1, 1.0, 0.0).astype(jnp.bfloat16)

    feat = feat_ref[pl.ds(s * BS, BS), :]
    a8 = a8_ref[...]
    ct = ct_ref[...]
    one = jnp.bfloat16(1.0)
    for h in range(N_HEAD):
        t = lax.dot_general(
            a8[:, 8 * h:8 * h + 8], ct[8 * h:8 * h + 8, :],
            (((1,), (0,)), ((), ())),
            preferred_element_type=jnp.float32).astype(jnp.bfloat16)
        p = jnp.maximum(t, one) * maskb
        acc_ref[:, h * HG:(h + 1) * HG] += lax.dot_general(
            p, feat[:, h * HG:(h + 1) * HG],
            (((0,), (0,)), ((), ())), preferred_element_type=jnp.float32)

    @pl.when(s == ns - 1)
    def _finalize():
        parts = []
        for h in range(N_HEAD):
            d = acc_ref[:, h * HG + OUT_C:h * HG + OUT_C + 1]
            parts.append(acc_ref[:, h * HG:h * HG + OUT_C]
                         / jnp.maximum(d, 1e-16))
        r = jnp.concatenate(parts, axis=1) + bias_ref[...]
        out_ref[...] = jnp.where(r > 0, r, jnp.exp(r) - 1.0)


@jax.jit
def kernel(lncrna_x, disease_x, adj, W, attn_l, attn_r, bias):
    z = jnp.concatenate([lncrna_x, disease_x], axis=0)
    zp = jnp.pad(z, ((0, NP - N), (0, 0)))

    eye = jnp.eye(N_HEAD, 8, dtype=jnp.float32)
    a_l = (attn_l[:, :, None] * eye[:, None, :]).reshape(FA, 8)
    a_r = (attn_r[:, :, None] * eye[:, None, :]).reshape(FA, 8)

    feat, a8_e, ct_e = pl.pallas_call(
        _prologue_body,
        grid=(NP // BP,),
        in_specs=[
            pl.BlockSpec((BP, IN_C), lambda i: (i, 0)),
            pl.BlockSpec((IN_C, FA), lambda i: (0, 0)),
            pl.BlockSpec((FA, 8), lambda i: (0, 0)),
            pl.BlockSpec((FA, 8), lambda i: (0, 0)),
        ],
        out_specs=[
            pl.BlockSpec((BP, AUGC), lambda i: (i, 0)),
            pl.BlockSpec((BP, 32), lambda i: (i, 0)),
            pl.BlockSpec((32, BP), lambda i: (0, i)),
        ],
        out_shape=[
            jax.ShapeDtypeStruct((NP, AUGC), jnp.bfloat16),
            jax.ShapeDtypeStruct((NP, 32), jnp.bfloat16),
            jax.ShapeDtypeStruct((32, NP), jnp.bfloat16),
        ],
    )(zp, W, a_l, a_r)

    nd, ns = NP // BD, NP // BS
    out = pl.pallas_call(
        functools.partial(_main_body, ns=ns),
        grid=(nd, ns),
        in_specs=[
            pl.BlockSpec((BS, BD), lambda d, s: (s, d)),
            pl.BlockSpec((NP, AUGC), lambda d, s: (0, 0)),
            pl.BlockSpec((BS, 32), lambda d, s: (s, 0)),
            pl.BlockSpec((32, BD), lambda d, s: (0, d)),
            pl.BlockSpec((1, FA), lambda d, s: (0, 0)),
        ],
        out_specs=pl.BlockSpec((BD, FA), lambda d, s: (d, 0)),
        out_shape=jax.ShapeDtypeStruct((NP, FA), jnp.float32),
        scratch_shapes=[
            pltpu.VMEM((BD, AUGC), jnp.float32),
        ],
        compiler_params=pltpu.CompilerParams(
            dimension_semantics=("parallel", "arbitrary")),
    )(adj, feat, a8_e, ct_e, bias.reshape(1, FA))
    return out[:N]


# speedup vs baseline: 1.9878x; 1.0268x over previous
import functools

import jax
import jax.numpy as jnp
from jax import lax
from jax.experimental import pallas as pl
from jax.experimental.pallas import tpu as pltpu

LNC = 5000
DIS = 3000
N = LNC + DIS
IN_C = 128
OUT_C = 64
N_HEAD = 4
NEG_SLOPE = 0.2

NP = 8192
BP = 512
BS = 2048
BD = 1024
FA = N_HEAD * OUT_C
HG = OUT_C + 8
AUGC = N_HEAD * HG


def _prologue_body(z_ref, w_ref, al_ref, ar_ref,
                   feat_ref, a8_ref, ct_ref):
    i = pl.program_id(0)
    z = z_ref[...]
    featf = jnp.dot(z, w_ref[...], preferred_element_type=jnp.float32)
    el = jnp.dot(featf, al_ref[...], preferred_element_type=jnp.float32)
    ert = lax.dot_general(
        ar_ref[...], featf, (((0,), (1,)), ((), ())),
        preferred_element_type=jnp.float32)
    a8 = jnp.exp((1.0 - NEG_SLOPE) * el)
    ct = jnp.exp((1.0 - NEG_SLOPE) * ert)
    zc = jnp.zeros((BP, 7), jnp.float32)
    a8_parts = []
    for h in range(N_HEAD):
        a8_parts.append(a8[:, h:h + 1])
        a8_parts.append(zc)
    a8_ref[...] = jnp.concatenate(a8_parts, axis=1).astype(jnp.bfloat16)
    zr = jnp.zeros((7, BP), jnp.float32)
    ct_parts = []
    for h in range(N_HEAD):
        ct_parts.append(ct[h:h + 1, :])
        ct_parts.append(zr)
    ct_ref[...] = jnp.concatenate(ct_parts, axis=0).astype(jnp.bfloat16)
    rowid = lax.broadcasted_iota(jnp.int32, (BP, 1), 0) + i * BP
    valid = (rowid < N).astype(jnp.float32)
    a2 = jnp.exp(NEG_SLOPE * el) * valid
    parts = []
    for h in range(N_HEAD):
        a2h = a2[:, h:h + 1]
        parts.append(featf[:, h * OUT_C:(h + 1) * OUT_C] * a2h)
        parts.append(jnp.broadcast_to(a2h, (BP, 8)))
    feat_ref[...] = jnp.concatenate(parts, axis=1).astype(jnp.bfloat16)


def _main_body(adj_ref, feat_ref, a8_ref, ct_ref, bias_ref, out_ref,
               acc_ref, *, ns):
    s = pl.program_id(1)

    @pl.when(s == 0)
    def _init():
        acc_ref[...] = jnp.zeros_like(acc_ref)

    adj = adj_ref[...]
    maskb = jnp.where(adj == 1, 1.0, 0.0).astype(jnp.bfloat16)

    feat = feat_ref[pl.ds(s * BS, BS), :]
    a8 = a8_ref[...]
    ct = ct_ref[...]
    one = jnp.bfloat16(1.0)
    for h in range(N_HEAD):
        t = lax.dot_general(
            a8[:, 8 * h:8 * h + 8], ct[8 * h:8 * h + 8, :],
            (((1,), (0,)), ((), ())),
            preferred_element_type=jnp.float32).astype(jnp.bfloat16)
        p = jnp.maximum(t, one) * maskb
        acc_ref[:, h * HG:(h + 1) * HG] += lax.dot_general(
            p, feat[:, h * HG:(h + 1) * HG],
            (((0,), (0,)), ((), ())), preferred_element_type=jnp.float32)

    @pl.when(s == ns - 1)
    def _finalize():
        parts = []
        for h in range(N_HEAD):
            d = acc_ref[:, h * HG + OUT_C:h * HG + OUT_C + 1]
            parts.append(acc_ref[:, h * HG:h * HG + OUT_C]
                         / jnp.maximum(d, 1e-16))
        r = jnp.concatenate(parts, axis=1) + bias_ref[...]
        out_ref[...] = jnp.where(r > 0, r, jnp.exp(r) - 1.0)


@jax.jit
def kernel(lncrna_x, disease_x, adj, W, attn_l, attn_r, bias):
    z = jnp.concatenate([lncrna_x, disease_x], axis=0)
    zp = jnp.pad(z, ((0, NP - N), (0, 0)))

    eye = jnp.eye(N_HEAD, 8, dtype=jnp.float32)
    a_l = (attn_l[:, :, None] * eye[:, None, :]).reshape(FA, 8)
    a_r = (attn_r[:, :, None] * eye[:, None, :]).reshape(FA, 8)

    feat, a8_e, ct_e = pl.pallas_call(
        _prologue_body,
        grid=(NP // BP,),
        in_specs=[
            pl.BlockSpec((BP, IN_C), lambda i: (i, 0)),
            pl.BlockSpec((IN_C, FA), lambda i: (0, 0)),
            pl.BlockSpec((FA, 8), lambda i: (0, 0)),
            pl.BlockSpec((FA, 8), lambda i: (0, 0)),
        ],
        out_specs=[
            pl.BlockSpec((BP, AUGC), lambda i: (i, 0)),
            pl.BlockSpec((BP, 32), lambda i: (i, 0)),
            pl.BlockSpec((32, BP), lambda i: (0, i)),
        ],
        out_shape=[
            jax.ShapeDtypeStruct((NP, AUGC), jnp.bfloat16),
            jax.ShapeDtypeStruct((NP, 32), jnp.bfloat16),
            jax.ShapeDtypeStruct((32, NP), jnp.bfloat16),
        ],
    )(zp, W, a_l, a_r)

    nd, ns = NP // BD, NP // BS
    out = pl.pallas_call(
        functools.partial(_main_body, ns=ns),
        grid=(nd, ns),
        in_specs=[
            pl.BlockSpec((BS, BD), lambda d, s: (s, d)),
            pl.BlockSpec((NP, AUGC), lambda d, s: (0, 0)),
            pl.BlockSpec((BS, 32), lambda d, s: (s, 0)),
            pl.BlockSpec((32, BD), lambda d, s: (0, d)),
            pl.BlockSpec((1, FA), lambda d, s: (0, 0)),
        ],
        out_specs=pl.BlockSpec((BD, FA), lambda d, s: (d, 0)),
        out_shape=jax.ShapeDtypeStruct((NP, FA), jnp.float32),
        scratch_shapes=[
            pltpu.VMEM((BD, AUGC), jnp.float32),
        ],
        compiler_params=pltpu.CompilerParams(
            dimension_semantics=("parallel", "arbitrary")),
    )(adj, feat, a8_e, ct_e, bias.reshape(1, FA))
    return out[:N]


# speedup vs baseline: 2.0570x; 1.0348x over previous
import functools

import jax
import jax.numpy as jnp
from jax import lax
from jax.experimental import pallas as pl
from jax.experimental.pallas import tpu as pltpu

LNC = 5000
DIS = 3000
N = LNC + DIS
IN_C = 128
OUT_C = 64
N_HEAD = 4
NEG_SLOPE = 0.2

NP = 8192
BP = 512
BS = 4096
BD = 512
FA = N_HEAD * OUT_C
HG = OUT_C + 8
AUGC = N_HEAD * HG


def _prologue_body(z_ref, w_ref, al_ref, ar_ref,
                   feat_ref, a8_ref, ct_ref):
    i = pl.program_id(0)
    z = z_ref[...]
    featf = jnp.dot(z, w_ref[...], preferred_element_type=jnp.float32)
    el = jnp.dot(featf, al_ref[...], preferred_element_type=jnp.float32)
    ert = lax.dot_general(
        ar_ref[...], featf, (((0,), (1,)), ((), ())),
        preferred_element_type=jnp.float32)
    a8 = jnp.exp((1.0 - NEG_SLOPE) * el)
    ct = jnp.exp((1.0 - NEG_SLOPE) * ert)
    zc = jnp.zeros((BP, 7), jnp.float32)
    a8_parts = []
    for h in range(N_HEAD):
        a8_parts.append(a8[:, h:h + 1])
        a8_parts.append(zc)
    a8_ref[...] = jnp.concatenate(a8_parts, axis=1).astype(jnp.bfloat16)
    zr = jnp.zeros((7, BP), jnp.float32)
    ct_parts = []
    for h in range(N_HEAD):
        ct_parts.append(ct[h:h + 1, :])
        ct_parts.append(zr)
    ct_ref[...] = jnp.concatenate(ct_parts, axis=0).astype(jnp.bfloat16)
    rowid = lax.broadcasted_iota(jnp.int32, (BP, 1), 0) + i * BP
    valid = (rowid < N).astype(jnp.float32)
    a2 = jnp.exp(NEG_SLOPE * el) * valid
    parts = []
    for h in range(N_HEAD):
        a2h = a2[:, h:h + 1]
        parts.append(featf[:, h * OUT_C:(h + 1) * OUT_C] * a2h)
        parts.append(jnp.broadcast_to(a2h, (BP, 8)))
    feat_ref[...] = jnp.concatenate(parts, axis=1).astype(jnp.bfloat16)


def _main_body(adj_ref, feat_ref, a8_ref, ct_ref, bias_ref, out_ref,
               acc_ref, *, ns):
    s = pl.program_id(1)

    @pl.when(s == 0)
    def _init():
        acc_ref[...] = jnp.zeros_like(acc_ref)

    adj = adj_ref[...]
    maskb = jnp.where(adj == 1, 1.0, 0.0).astype(jnp.bfloat16)

    feat = feat_ref[pl.ds(s * BS, BS), :]
    a8 = a8_ref[...]
    ct = ct_ref[...]
    one = jnp.bfloat16(1.0)
    for h in range(N_HEAD):
        t = lax.dot_general(
            a8[:, 8 * h:8 * h + 8], ct[8 * h:8 * h + 8, :],
            (((1,), (0,)), ((), ())),
            preferred_element_type=jnp.float32).astype(jnp.bfloat16)
        p = jnp.maximum(t, one) * maskb
        acc_ref[:, h * HG:(h + 1) * HG] += lax.dot_general(
            p, feat[:, h * HG:(h + 1) * HG],
            (((0,), (0,)), ((), ())), preferred_element_type=jnp.float32)

    @pl.when(s == ns - 1)
    def _finalize():
        parts = []
        for h in range(N_HEAD):
            d = acc_ref[:, h * HG + OUT_C:h * HG + OUT_C + 1]
            parts.append(acc_ref[:, h * HG:h * HG + OUT_C]
                         / jnp.maximum(d, 1e-16))
        r = jnp.concatenate(parts, axis=1) + bias_ref[...]
        out_ref[...] = jnp.where(r > 0, r, jnp.exp(r) - 1.0)


@jax.jit
def kernel(lncrna_x, disease_x, adj, W, attn_l, attn_r, bias):
    z = jnp.concatenate([lncrna_x, disease_x], axis=0)
    zp = jnp.pad(z, ((0, NP - N), (0, 0)))

    eye = jnp.eye(N_HEAD, 8, dtype=jnp.float32)
    a_l = (attn_l[:, :, None] * eye[:, None, :]).reshape(FA, 8)
    a_r = (attn_r[:, :, None] * eye[:, None, :]).reshape(FA, 8)

    feat, a8_e, ct_e = pl.pallas_call(
        _prologue_body,
        grid=(NP // BP,),
        in_specs=[
            pl.BlockSpec((BP, IN_C), lambda i: (i, 0)),
            pl.BlockSpec((IN_C, FA), lambda i: (0, 0)),
            pl.BlockSpec((FA, 8), lambda i: (0, 0)),
            pl.BlockSpec((FA, 8), lambda i: (0, 0)),
        ],
        out_specs=[
            pl.BlockSpec((BP, AUGC), lambda i: (i, 0)),
            pl.BlockSpec((BP, 32), lambda i: (i, 0)),
            pl.BlockSpec((32, BP), lambda i: (0, i)),
        ],
        out_shape=[
            jax.ShapeDtypeStruct((NP, AUGC), jnp.bfloat16),
            jax.ShapeDtypeStruct((NP, 32), jnp.bfloat16),
            jax.ShapeDtypeStruct((32, NP), jnp.bfloat16),
        ],
    )(zp, W, a_l, a_r)

    nd, ns = NP // BD, NP // BS
    out = pl.pallas_call(
        functools.partial(_main_body, ns=ns),
        grid=(nd, ns),
        in_specs=[
            pl.BlockSpec((BS, BD), lambda d, s: (s, d)),
            pl.BlockSpec((NP, AUGC), lambda d, s: (0, 0)),
            pl.BlockSpec((BS, 32), lambda d, s: (s, 0)),
            pl.BlockSpec((32, BD), lambda d, s: (0, d)),
            pl.BlockSpec((1, FA), lambda d, s: (0, 0)),
        ],
        out_specs=pl.BlockSpec((BD, FA), lambda d, s: (d, 0)),
        out_shape=jax.ShapeDtypeStruct((NP, FA), jnp.float32),
        scratch_shapes=[
            pltpu.VMEM((BD, AUGC), jnp.float32),
        ],
        compiler_params=pltpu.CompilerParams(
            dimension_semantics=("parallel", "arbitrary")),
    )(adj, feat, a8_e, ct_e, bias.reshape(1, FA))
    return out[:N]
